# baseline (speedup 1.0000x reference)
import jax
import jax.numpy as jnp
from jax import lax
from jax.experimental import pallas as pl
from jax.experimental.pallas import tpu as pltpu
from jax.experimental.pallas import tpu_sc as plsc

_VOC = 1000000
_E = 64
_H = 128
_LN = 256
_NC = 10
_B = 4096
_S = 50

_NCORES = 2
_NSUB = 16
_NW = _NCORES * _NSUB
_NIDX = _B * _S
_BPW = _NIDX // _NW
_CH = 800
_NCH = _BPW // _CH


def _gather_body(table_hbm, idx_hbm, out_hbm, idx_v, rows_v, gsem):
    wid = lax.axis_index("s") * _NCORES + lax.axis_index("c")
    base = wid * _BPW
    pltpu.sync_copy(idx_hbm.at[pl.ds(base, _BPW)], idx_v)
    for j in range(_NCH):
        pltpu.async_copy(
            table_hbm.at[idx_v.at[pl.ds(j * _CH, _CH)]], rows_v, gsem
        ).wait()
        pltpu.sync_copy(rows_v, out_hbm.at[pl.ds(base + j * _CH, _CH)])


def _sc_gather(table_packed, pair_idx):
    mesh = plsc.VectorSubcoreMesh(core_axis_name="c", subcore_axis_name="s")
    f = pl.kernel(
        _gather_body,
        mesh=mesh,
        out_type=jax.ShapeDtypeStruct((_NIDX, 2 * _E), jnp.float32),
        scratch_types=[
            pltpu.VMEM((_BPW,), jnp.int32),
            pltpu.VMEM((_CH, 2 * _E), jnp.float32),
            pltpu.SemaphoreType.DMA,
        ],
    )
    return f(table_packed, pair_idx)


_BB = 4096
_NB = _B // _BB


def _lstm_body(ef_ref, eb_ref, pf_ref, pb_ref, idx_ref,
               wxf_ref, wxb_ref,
               bf_ref, bb_ref, w1_ref, b1_ref, w2_ref, b2_ref,
               out_ref, h_f, c_f, h_b, c_b, lf, lb):
    s = pl.program_id(1)

    @pl.when(s == 0)
    def _init():
        z0 = jnp.zeros((_BB, _H), jnp.float32)
        h_f[...] = z0
        c_f[...] = z0
        h_b[...] = z0
        c_b[...] = z0
        lf[...] = z0
        lb[...] = z0

    lane = lax.broadcasted_iota(jnp.int32, (_BB, 2 * _E), 1)
    hi_half = lane >= _E

    def _step(e_ref, p_ref, w_ref, b_ref, h, c):
        masked = jnp.where((p_ref[0] != 0) == hi_half, e_ref[0], 0.0)
        a = jnp.concatenate([masked, h[...]], axis=1)
        z = (jnp.dot(a, w_ref[...], preferred_element_type=jnp.float32)
             + b_ref[0:1, :])
        gi = jax.nn.sigmoid(z[:, 0 * _H:1 * _H])
        gf = jax.nn.sigmoid(z[:, 1 * _H:2 * _H])
        gg = jnp.tanh(z[:, 2 * _H:3 * _H])
        go = jax.nn.sigmoid(z[:, 3 * _H:4 * _H])
        cn = gf * c[...] + gi * gg
        hn = go * jnp.tanh(cn)
        h[...] = hn
        c[...] = cn
        return hn

    hf = _step(ef_ref, pf_ref, wxf_ref, bf_ref, h_f, c_f)
    hb = _step(eb_ref, pb_ref, wxb_ref, bb_ref, h_b, c_b)

    idxv = idx_ref[...]
    lf[...] = jnp.where(idxv == s, hf, lf[...])
    lb[...] = jnp.where(idxv == (_S - 1 - s), hb, lb[...])

    @pl.when(s == _S - 1)
    def _final():
        last = jnp.concatenate([lf[...], lb[...]], axis=1)
        x1 = (jnp.dot(last, w1_ref[...], preferred_element_type=jnp.float32)
              + b1_ref[0:1, :])
        out_ref[...] = (jnp.dot(x1, w2_ref[...], preferred_element_type=jnp.float32)
                        + b2_ref[0:1, :])


def _full(shape):
    return pl.BlockSpec(shape, lambda nb, s: tuple(0 for _ in shape))


def _lstm_call_kwargs():
    return dict(
        grid=(_NB, _S),
        in_specs=[
            pl.BlockSpec((1, _BB, 2 * _E), lambda nb, s: (s, nb, 0)),
            pl.BlockSpec((1, _BB, 2 * _E), lambda nb, s: (_S - 1 - s, nb, 0)),
            pl.BlockSpec((1, _BB, 1), lambda nb, s: (s, nb, 0)),
            pl.BlockSpec((1, _BB, 1), lambda nb, s: (_S - 1 - s, nb, 0)),
            pl.BlockSpec((_BB, _H), lambda nb, s: (nb, 0)),
            _full((2 * _H, 4 * _H)), _full((2 * _H, 4 * _H)),
            _full((8, 4 * _H)), _full((8, 4 * _H)),
            _full((2 * _H, _LN)), _full((8, _LN)),
            _full((_LN, _H)), _full((8, _H)),
        ],
        out_specs=pl.BlockSpec((_BB, _H), lambda nb, s: (nb, 0)),
        out_shape=jax.ShapeDtypeStruct((_B, _H), jnp.float32),
        scratch_shapes=[pltpu.VMEM((_BB, _H), jnp.float32)] * 6,
        compiler_params=pltpu.CompilerParams(
            dimension_semantics=("arbitrary", "arbitrary")),
    )


def _bilstm_head(emb, par, idxb, wf, wb, bf2, bb2, w1t, b1t, w2tp, b2p):
    return pl.pallas_call(_lstm_body, **_lstm_call_kwargs())(
        emb, emb, par, par, idxb, wf, wb, bf2, bb2, w1t, b1t, w2tp, b2p)


def kernel(x_inputs, len_sequences, embed_w, Wih_f, Whh_f, bih_f, bhh_f,
           Wih_b, Whh_b, bih_b, bhh_b, W1, b1, W2, b2):
    flat_idx = jnp.transpose(x_inputs).reshape(_NIDX)
    table_packed = jnp.concatenate([embed_w[0::2], embed_w[1::2]], axis=1)
    emb = _sc_gather(table_packed, flat_idx >> 1).reshape(_S, _B, 2 * _E)
    par = (flat_idx & 1).astype(jnp.int8).reshape(_S, _B, 1)

    idx = (jnp.clip(len_sequences, 1, _S) - 1).astype(jnp.int32)
    idxb = jnp.broadcast_to(idx[:, None], (_B, _H))

    wf = jnp.concatenate([Wih_f.T, Wih_f.T, Whh_f.T], axis=0)
    wb = jnp.concatenate([Wih_b.T, Wih_b.T, Whh_b.T], axis=0)
    bf2 = jnp.broadcast_to((bih_f + bhh_f)[None, :], (8, 4 * _H))
    bb2 = jnp.broadcast_to((bih_b + bhh_b)[None, :], (8, 4 * _H))
    w1t = W1.T
    b1t = jnp.broadcast_to(b1[None, :], (8, _LN))
    w2tp = jnp.zeros((_LN, _H), jnp.float32).at[:, :_NC].set(W2.T)
    b2p = jnp.broadcast_to(
        jnp.zeros((_H,), jnp.float32).at[:_NC].set(b2)[None, :], (8, _H))

    out_pad = _bilstm_head(emb, par, idxb, wf, wb,
                           bf2, bb2, w1t, b1t, w2tp, b2p)
    return out_pad[:, :_NC]


# speedup vs baseline: 9.6083x; 9.6083x over previous
import jax
import jax.numpy as jnp
from jax import lax
from jax.experimental import pallas as pl
from jax.experimental.pallas import tpu as pltpu
from jax.experimental.pallas import tpu_sc as plsc

_VOC = 1000000
_E = 64
_H = 128
_LN = 256
_NC = 10
_B = 4096
_S = 50

_NCORES = 2
_NSUB = 16
_NW = _NCORES * _NSUB
_NIDX = _B * _S
_BPW = _NIDX // _NW
_CH = 400
_NCH = _BPW // _CH


def _gather_body(table_hbm, idx_hbm, out_hbm, idx_v, rows_a, rows_b,
                 gsem, wsem):
    wid = lax.axis_index("s") * _NCORES + lax.axis_index("c")
    base = wid * _BPW
    pltpu.sync_copy(idx_hbm.at[pl.ds(base, _BPW)], idx_v)
    bufs = (rows_a, rows_b)
    g = pltpu.async_copy(
        table_hbm.at[idx_v.at[pl.ds(0, _CH)]], bufs[0], gsem)
    w_prev = None
    w_last = None
    for j in range(_NCH):
        g.wait()
        if j + 1 < _NCH:
            if w_prev is not None:
                w_prev.wait()
            g = pltpu.async_copy(
                table_hbm.at[idx_v.at[pl.ds((j + 1) * _CH, _CH)]],
                bufs[(j + 1) % 2], gsem)
            w_prev = pltpu.async_copy(
                bufs[j % 2], out_hbm.at[pl.ds(base + j * _CH, _CH)],
                wsem)
        else:
            w_last = pltpu.async_copy(
                bufs[j % 2], out_hbm.at[pl.ds(base + j * _CH, _CH)],
                wsem)
    if w_prev is not None:
        w_prev.wait()
    w_last.wait()


def _sc_gather(table_packed, pair_idx):
    mesh = plsc.VectorSubcoreMesh(core_axis_name="c", subcore_axis_name="s")
    f = pl.kernel(
        _gather_body,
        mesh=mesh,
        out_type=jax.ShapeDtypeStruct((_NIDX, 2 * _E), jnp.float32),
        scratch_types=[
            pltpu.VMEM((_BPW,), jnp.int32),
            pltpu.VMEM((_CH, 2 * _E), jnp.float32),
            pltpu.VMEM((_CH, 2 * _E), jnp.float32),
            pltpu.SemaphoreType.DMA,
            pltpu.SemaphoreType.DMA,
        ],
    )
    return f(table_packed, pair_idx)


_BB = 4096
_NB = _B // _BB


def _lstm_body(ef_ref, eb_ref, pf_ref, pb_ref, idx_ref,
               wxf_ref, wxb_ref,
               bf_ref, bb_ref, w1_ref, b1_ref, w2_ref, b2_ref,
               out_ref, h_f, c_f, h_b, c_b, lf, lb):
    s = pl.program_id(1)

    @pl.when(s == 0)
    def _init():
        z0 = jnp.zeros((_BB, _H), jnp.float32)
        h_f[...] = z0
        c_f[...] = z0
        h_b[...] = z0
        c_b[...] = z0
        lf[...] = z0
        lb[...] = z0

    lane = lax.broadcasted_iota(jnp.int32, (_BB, 2 * _E), 1)
    hi_half = lane >= _E

    def _step(e_ref, p_ref, w_ref, b_ref, h, c):
        masked = jnp.where((p_ref[0] != 0) == hi_half, e_ref[0], 0.0)
        a = jnp.concatenate([masked, h[...]], axis=1)
        z = (jnp.dot(a, w_ref[...], preferred_element_type=jnp.float32)
             + b_ref[0:1, :])
        gi = jax.nn.sigmoid(z[:, 0 * _H:1 * _H])
        gf = jax.nn.sigmoid(z[:, 1 * _H:2 * _H])
        gg = jnp.tanh(z[:, 2 * _H:3 * _H])
        go = jax.nn.sigmoid(z[:, 3 * _H:4 * _H])
        cn = gf * c[...] + gi * gg
        hn = go * jnp.tanh(cn)
        h[...] = hn
        c[...] = cn
        return hn

    hf = _step(ef_ref, pf_ref, wxf_ref, bf_ref, h_f, c_f)
    hb = _step(eb_ref, pb_ref, wxb_ref, bb_ref, h_b, c_b)

    idxv = idx_ref[...]
    lf[...] = jnp.where(idxv == s, hf, lf[...])
    lb[...] = jnp.where(idxv == (_S - 1 - s), hb, lb[...])

    @pl.when(s == _S - 1)
    def _final():
        last = jnp.concatenate([lf[...], lb[...]], axis=1)
        x1 = (jnp.dot(last, w1_ref[...], preferred_element_type=jnp.float32)
              + b1_ref[0:1, :])
        out_ref[...] = (jnp.dot(x1, w2_ref[...], preferred_element_type=jnp.float32)
                        + b2_ref[0:1, :])


def _full(shape):
    return pl.BlockSpec(shape, lambda nb, s: tuple(0 for _ in shape))


def _lstm_call_kwargs():
    return dict(
        grid=(_NB, _S),
        in_specs=[
            pl.BlockSpec((1, _BB, 2 * _E), lambda nb, s: (s, nb, 0)),
            pl.BlockSpec((1, _BB, 2 * _E), lambda nb, s: (_S - 1 - s, nb, 0)),
            pl.BlockSpec((1, _BB, 1), lambda nb, s: (s, nb, 0)),
            pl.BlockSpec((1, _BB, 1), lambda nb, s: (_S - 1 - s, nb, 0)),
            pl.BlockSpec((_BB, _H), lambda nb, s: (nb, 0)),
            _full((2 * _H, 4 * _H)), _full((2 * _H, 4 * _H)),
            _full((8, 4 * _H)), _full((8, 4 * _H)),
            _full((2 * _H, _LN)), _full((8, _LN)),
            _full((_LN, _H)), _full((8, _H)),
        ],
        out_specs=pl.BlockSpec((_BB, _H), lambda nb, s: (nb, 0)),
        out_shape=jax.ShapeDtypeStruct((_B, _H), jnp.float32),
        scratch_shapes=[pltpu.VMEM((_BB, _H), jnp.float32)] * 6,
        compiler_params=pltpu.CompilerParams(
            dimension_semantics=("arbitrary", "arbitrary")),
    )


def _bilstm_head(emb, par, idxb, wf, wb, bf2, bb2, w1t, b1t, w2tp, b2p):
    return pl.pallas_call(_lstm_body, **_lstm_call_kwargs())(
        emb, emb, par, par, idxb, wf, wb, bf2, bb2, w1t, b1t, w2tp, b2p)


def kernel(x_inputs, len_sequences, embed_w, Wih_f, Whh_f, bih_f, bhh_f,
           Wih_b, Whh_b, bih_b, bhh_b, W1, b1, W2, b2):
    flat_idx = jnp.transpose(x_inputs).reshape(_NIDX)
    table_packed = embed_w.reshape(_VOC // 2, 2 * _E)
    emb = _sc_gather(table_packed, flat_idx >> 1).reshape(_S, _B, 2 * _E)
    par = (flat_idx & 1).astype(jnp.int8).reshape(_S, _B, 1)

    idx = (jnp.clip(len_sequences, 1, _S) - 1).astype(jnp.int32)
    idxb = jnp.broadcast_to(idx[:, None], (_B, _H))

    wf = jnp.concatenate([Wih_f.T, Wih_f.T, Whh_f.T], axis=0)
    wb = jnp.concatenate([Wih_b.T, Wih_b.T, Whh_b.T], axis=0)
    bf2 = jnp.broadcast_to((bih_f + bhh_f)[None, :], (8, 4 * _H))
    bb2 = jnp.broadcast_to((bih_b + bhh_b)[None, :], (8, 4 * _H))
    w1t = W1.T
    b1t = jnp.broadcast_to(b1[None, :], (8, _LN))
    w2tp = jnp.zeros((_LN, _H), jnp.float32).at[:, :_NC].set(W2.T)
    b2p = jnp.broadcast_to(
        jnp.zeros((_H,), jnp.float32).at[:_NC].set(b2)[None, :], (8, _H))

    out_pad = _bilstm_head(emb, par, idxb, wf, wb,
                           bf2, bb2, w1t, b1t, w2tp, b2p)
    return out_pad[:, :_NC]
